# baseline (speedup 1.0000x reference)
import jax
import jax.numpy as jnp
from jax import lax
from jax.experimental import pallas as pl
from jax.experimental.pallas import tpu as pltpu
from jax.experimental.pallas import tpu_sc as plsc

_EMB = 64
_SCALE = 8.0

_B = 4096
_L = 200
_N = _B * _L
_NW = 32
_NBH = _B // 128
_UPW = (_L * _NBH) // 2 // _NW
_TPU_ROWS = 2 * _UPW


def _sc_embed(tok_hbm, table_hbm, out_hbm,
              idx_all, rows0, rows1, ob0, ob1, gsem, osem):
    wid = lax.axis_index("s") * 2 + lax.axis_index("c")
    u0 = wid * _UPW
    trow0 = wid * _TPU_ROWS

    pltpu.sync_copy(tok_hbm.at[pl.ds(trow0, _TPU_ROWS)], idx_all)

    rows = (rows0, rows1)
    obs = (ob0, ob1)
    iota16 = lax.iota(jnp.int32, 16)
    rot = [lax.rem(iota16 + d, 16) for d in range(16)]

    def start_gather(c, b):
        for j in range(2):
            pltpu.async_copy(
                table_hbm.at[idx_all.at[c * 2 + j]],
                rows[b].at[pl.ds(j * 128, 128)],
                gsem.at[b],
            )

    def drain_gather(b):
        pltpu.make_async_copy(
            table_hbm.at[pl.ds(0, 256)], rows[b], gsem.at[b]
        ).wait()

    def drain_out(b):
        pltpu.make_async_copy(
            rows[b], table_hbm.at[pl.ds(0, 256)], osem.at[b]
        ).wait()

    def transpose_scale(b):
        rv = rows[b]
        ob = obs[b]

        for bl in range(2):
            obl = ob.at[bl]

            @plsc.parallel_loop(0, 32, unroll=2)
            def blkbody(u2, obl=obl, bl=bl):
                tb = lax.shift_right_logical(u2, 2)
                eb = lax.bitwise_and(u2, 3)
                t0 = bl * 128 + tb * 16
                blo0 = tb * 16
                e0 = eb * 16
                row_ids = iota16 + t0
                blo_ids = iota16 + blo0
                for h in range(2):
                    e_ids = [rot[8 * h + d] + e0 for d in range(8)]
                    vals = [plsc.load_gather(rv, [row_ids, e])
                            for e in e_ids]
                    for d in range(8):
                        plsc.store_scatter(
                            obl, [e_ids[d], blo_ids], vals[d] * _SCALE
                        )

    def start_out(c, b):
        u = u0 + c
        l = u // 16
        g = lax.rem(u, 16)
        for bl in range(2):
            for ehi in range(8):
                pltpu.async_copy(
                    obs[b].at[bl, pl.ds(8 * ehi, 8)],
                    out_hbm.at[l, ehi, g * 2 + bl],
                    osem.at[b],
                )

    start_gather(0, 0)

    def step(k, carry):
        for b in range(2):
            c = k * 2 + b
            b2 = 1 - b
            drain_gather(b)

            @pl.when(c + 1 < _UPW)
            def _prefetch():
                @pl.when(c >= 1)
                def _free_buf():
                    drain_out(b2)
                start_gather(c + 1, b2)

            transpose_scale(b)
            start_out(c, b)
        return carry

    lax.fori_loop(0, _UPW // 2, step, 0)
    drain_out(0)
    drain_out(1)


def kernel(tokens, table):
    tok2d = tokens.astype(jnp.int32).T.reshape(_L * _NBH, 128)
    mesh = plsc.VectorSubcoreMesh(core_axis_name="c", subcore_axis_name="s")
    out5 = pl.kernel(
        _sc_embed,
        out_type=jax.ShapeDtypeStruct((_L, 8, _NBH, 8, 128), jnp.float32),
        mesh=mesh,
        scratch_types=[
            pltpu.VMEM((_TPU_ROWS, 128), jnp.int32),
            pltpu.VMEM((256, _EMB), jnp.float32),
            pltpu.VMEM((256, _EMB), jnp.float32),
            pltpu.VMEM((2, _EMB, 128), jnp.float32),
            pltpu.VMEM((2, _EMB, 128), jnp.float32),
            pltpu.SemaphoreType.DMA((2,)),
            pltpu.SemaphoreType.DMA((2,)),
        ],
        compiler_params=pltpu.CompilerParams(
            use_tc_tiling_on_sc=False, needs_layout_passes=False
        ),
    )(tok2d, table)
    return out5.transpose((2, 4, 0, 1, 3)).reshape(_B, _L, _EMB)


# speedup vs baseline: 1.0217x; 1.0217x over previous
import jax
import jax.numpy as jnp
from jax import lax
from jax.experimental import pallas as pl
from jax.experimental.pallas import tpu as pltpu
from jax.experimental.pallas import tpu_sc as plsc

_EMB = 64
_SCALE = 8.0

_B = 4096
_L = 200
_N = _B * _L
_NW = 32
_NBH = _B // 128
_UPW = (_L * _NBH) // 2 // _NW
_TPU_ROWS = 2 * _UPW


def _sc_embed(tok_hbm, table_hbm, out_hbm,
              idx_all, rows0, rows1, ob0, ob1, gsem, osem):
    wid = lax.axis_index("s") * 2 + lax.axis_index("c")
    u0 = wid * _UPW
    trow0 = wid * _TPU_ROWS

    pltpu.sync_copy(tok_hbm.at[pl.ds(trow0, _TPU_ROWS)], idx_all)

    rows = (rows0, rows1)
    obs = (ob0, ob1)
    iota16 = lax.iota(jnp.int32, 16)
    rot = [lax.rem(iota16 + d, 16) for d in range(16)]

    def start_gather(c, b):
        for j in range(2):
            pltpu.async_copy(
                table_hbm.at[idx_all.at[c * 2 + j]],
                rows[b].at[pl.ds(j * 128, 128)],
                gsem.at[b],
            )

    def drain_gather(b):
        pltpu.make_async_copy(
            table_hbm.at[pl.ds(0, 256)], rows[b], gsem.at[b]
        ).wait()

    def drain_out(b):
        pltpu.make_async_copy(
            rows[b], table_hbm.at[pl.ds(0, 256)], osem.at[b]
        ).wait()

    def transpose_scale(b):
        rv = rows[b]
        ob = obs[b]

        for bl in range(2):
            obl = ob.at[bl]

            def blkbody(u2, carry, obl=obl, bl=bl):
                tb = lax.shift_right_logical(u2, 2)
                eb = lax.bitwise_and(u2, 3)
                t0 = bl * 128 + tb * 16
                blo0 = tb * 16
                e0 = eb * 16
                row_ids = iota16 + t0
                blo_ids = iota16 + blo0
                for h in range(2):
                    e_ids = [rot[8 * h + d] + e0 for d in range(8)]
                    vals = [plsc.load_gather(rv, [row_ids, e])
                            for e in e_ids]
                    for d in range(8):
                        plsc.store_scatter(
                            obl, [e_ids[d], blo_ids], vals[d] * _SCALE
                        )
                return carry

            lax.fori_loop(0, 32, blkbody, 0)

    def start_out(c, b):
        u = u0 + c
        l = u // 16
        g = lax.rem(u, 16)
        for bl in range(2):
            for ehi in range(8):
                pltpu.async_copy(
                    obs[b].at[bl, pl.ds(8 * ehi, 8)],
                    out_hbm.at[l, ehi, g * 2 + bl],
                    osem.at[b],
                )

    start_gather(0, 0)

    def step(k, carry):
        for b in range(2):
            c = k * 2 + b
            b2 = 1 - b
            drain_gather(b)

            @pl.when(c + 1 < _UPW)
            def _prefetch():
                @pl.when(c >= 1)
                def _free_buf():
                    drain_out(b2)
                start_gather(c + 1, b2)

            transpose_scale(b)
            start_out(c, b)
        return carry

    lax.fori_loop(0, _UPW // 2, step, 0)
    drain_out(0)
    drain_out(1)


def kernel(tokens, table):
    tok2d = tokens.astype(jnp.int32).T.reshape(_L * _NBH, 128)
    mesh = plsc.VectorSubcoreMesh(core_axis_name="c", subcore_axis_name="s")
    out5 = pl.kernel(
        _sc_embed,
        out_type=jax.ShapeDtypeStruct((_L, 8, _NBH, 8, 128), jnp.float32),
        mesh=mesh,
        scratch_types=[
            pltpu.VMEM((_TPU_ROWS, 128), jnp.int32),
            pltpu.VMEM((256, _EMB), jnp.float32),
            pltpu.VMEM((256, _EMB), jnp.float32),
            pltpu.VMEM((2, _EMB, 128), jnp.float32),
            pltpu.VMEM((2, _EMB, 128), jnp.float32),
            pltpu.SemaphoreType.DMA((2,)),
            pltpu.SemaphoreType.DMA((2,)),
        ],
        compiler_params=pltpu.CompilerParams(
            use_tc_tiling_on_sc=False, needs_layout_passes=False
        ),
    )(tok2d, table)
    return out5.transpose((2, 4, 0, 1, 3)).reshape(_B, _L, _EMB)


# speedup vs baseline: 1.0315x; 1.0096x over previous
import jax
import jax.numpy as jnp
from jax import lax
from jax.experimental import pallas as pl
from jax.experimental.pallas import tpu as pltpu
from jax.experimental.pallas import tpu_sc as plsc

_EMB = 64
_SCALE = 8.0

_B = 4096
_L = 200
_N = _B * _L
_NW = 32
_NBH = _B // 128
_UPW = (_L * _NBH) // 2 // _NW
_TPU_ROWS = 2 * _UPW


def _sc_embed(tok_hbm, table_hbm, out_hbm,
              idx_all, rows0, rows1, ob0, ob1, gsem, osem):
    wid = lax.axis_index("s") * 2 + lax.axis_index("c")
    u0 = wid * _UPW
    trow0 = wid * _TPU_ROWS

    pltpu.sync_copy(tok_hbm.at[pl.ds(trow0, _TPU_ROWS)], idx_all)

    rows = (rows0, rows1)
    obs = (ob0, ob1)
    iota16 = lax.iota(jnp.int32, 16)
    rot = [lax.rem(iota16 + d, 16) for d in range(16)]

    def start_gather(c, b):
        for j in range(2):
            pltpu.async_copy(
                table_hbm.at[idx_all.at[c * 2 + j]],
                rows[b].at[pl.ds(j * 128, 128)],
                gsem.at[b],
            )

    def drain_gather(b):
        pltpu.make_async_copy(
            table_hbm.at[pl.ds(0, 256)], rows[b], gsem.at[b]
        ).wait()

    def drain_out(b):
        pltpu.make_async_copy(
            rows[b], table_hbm.at[pl.ds(0, 256)], osem.at[b]
        ).wait()

    def transpose_scale(b):
        rv = rows[b]
        ob = obs[b]

        for bl in range(2):
            obl = ob.at[bl]

            def blkbody(u2, carry, obl=obl, bl=bl):
                tb = lax.shift_right_logical(u2, 2)
                eb = lax.bitwise_and(u2, 3)
                t0 = bl * 128 + tb * 16
                blo0 = tb * 16
                e0 = eb * 16
                row_ids = iota16 + t0
                blo_ids = iota16 + blo0
                for h in range(1):
                    e_ids = [rot[d] + e0 for d in range(16)]
                    vals = [plsc.load_gather(rv, [row_ids, e])
                            for e in e_ids]
                    for d in range(16):
                        plsc.store_scatter(
                            obl, [e_ids[d], blo_ids], vals[d] * _SCALE
                        )
                return carry

            lax.fori_loop(0, 32, blkbody, 0)

    def start_out(c, b):
        u = u0 + c
        l = u // 16
        g = lax.rem(u, 16)
        for bl in range(2):
            for ehi in range(8):
                pltpu.async_copy(
                    obs[b].at[bl, pl.ds(8 * ehi, 8)],
                    out_hbm.at[l, ehi, g * 2 + bl],
                    osem.at[b],
                )

    start_gather(0, 0)

    def step(k, carry):
        for b in range(2):
            c = k * 2 + b
            b2 = 1 - b
            drain_gather(b)

            @pl.when(c + 1 < _UPW)
            def _prefetch():
                @pl.when(c >= 1)
                def _free_buf():
                    drain_out(b2)
                start_gather(c + 1, b2)

            transpose_scale(b)
            start_out(c, b)
        return carry

    lax.fori_loop(0, _UPW // 2, step, 0)
    drain_out(0)
    drain_out(1)


def kernel(tokens, table):
    tok2d = tokens.astype(jnp.int32).T.reshape(_L * _NBH, 128)
    mesh = plsc.VectorSubcoreMesh(core_axis_name="c", subcore_axis_name="s")
    out5 = pl.kernel(
        _sc_embed,
        out_type=jax.ShapeDtypeStruct((_L, 8, _NBH, 8, 128), jnp.float32),
        mesh=mesh,
        scratch_types=[
            pltpu.VMEM((_TPU_ROWS, 128), jnp.int32),
            pltpu.VMEM((256, _EMB), jnp.float32),
            pltpu.VMEM((256, _EMB), jnp.float32),
            pltpu.VMEM((2, _EMB, 128), jnp.float32),
            pltpu.VMEM((2, _EMB, 128), jnp.float32),
            pltpu.SemaphoreType.DMA((2,)),
            pltpu.SemaphoreType.DMA((2,)),
        ],
        compiler_params=pltpu.CompilerParams(
            use_tc_tiling_on_sc=False, needs_layout_passes=False
        ),
    )(tok2d, table)
    return out5.transpose((2, 4, 0, 1, 3)).reshape(_B, _L, _EMB)


# speedup vs baseline: 1.1108x; 1.0768x over previous
import jax
import jax.numpy as jnp
from jax import lax
from jax.experimental import pallas as pl
from jax.experimental.pallas import tpu as pltpu
from jax.experimental.pallas import tpu_sc as plsc

_EMB = 64
_SCALE = 8.0

_B = 4096
_L = 200
_N = _B * _L
_NW = 32
_NBH = _B // 128
_UPW = (_L * _NBH) // 2 // _NW
_TPU_ROWS = 2 * _UPW


def _sc_embed(tok_hbm, table_hbm, out_hbm,
              idx_all, rows0, rows1, rows2, ob0, ob1, ob2, gsem, osem):
    wid = lax.axis_index("s") * 2 + lax.axis_index("c")
    u0 = wid * _UPW
    trow0 = wid * _TPU_ROWS

    pltpu.sync_copy(tok_hbm.at[pl.ds(trow0, _TPU_ROWS)], idx_all)

    rows = (rows0, rows1, rows2)
    obs = (ob0, ob1, ob2)
    iota16 = lax.iota(jnp.int32, 16)
    rot = [lax.rem(iota16 + d, 16) for d in range(16)]

    def start_gather(c, b):
        for j in range(2):
            pltpu.async_copy(
                table_hbm.at[idx_all.at[c * 2 + j]],
                rows[b].at[pl.ds(j * 128, 128)],
                gsem.at[b],
            )

    def drain_gather(b):
        pltpu.make_async_copy(
            table_hbm.at[pl.ds(0, 256)], rows[b], gsem.at[b]
        ).wait()

    def drain_out(b):
        pltpu.make_async_copy(
            rows[b], table_hbm.at[pl.ds(0, 256)], osem.at[b]
        ).wait()

    def transpose_scale(b):
        rv = rows[b]
        ob = obs[b]

        for bl in range(2):
            obl = ob.at[bl]

            def blkbody(u2, carry, obl=obl, bl=bl):
                tb = lax.shift_right_logical(u2, 2)
                eb = lax.bitwise_and(u2, 3)
                t0 = bl * 128 + tb * 16
                blo0 = tb * 16
                e0 = eb * 16
                row_ids = iota16 + t0
                blo_ids = iota16 + blo0
                for h in range(1):
                    e_ids = [rot[d] + e0 for d in range(16)]
                    vals = [plsc.load_gather(rv, [row_ids, e])
                            for e in e_ids]
                    for d in range(16):
                        plsc.store_scatter(
                            obl, [e_ids[d], blo_ids], vals[d] * _SCALE
                        )
                return carry

            lax.fori_loop(0, 32, blkbody, 0)

    def start_out(c, b):
        u = u0 + c
        l = u // 16
        g = lax.rem(u, 16)
        for bl in range(2):
            for ehi in range(8):
                pltpu.async_copy(
                    obs[b].at[bl, pl.ds(8 * ehi, 8)],
                    out_hbm.at[l, ehi, g * 2 + bl],
                    osem.at[b],
                )

    start_gather(0, 0)
    start_gather(1, 1)

    def step(k, carry):
        for b in range(3):
            c = k * 3 + b

            @pl.when(c < _UPW)
            def _unit():
                drain_gather(b)

                @pl.when(c + 2 < _UPW)
                def _prefetch():
                    start_gather(c + 2, (b + 2) % 3)

                @pl.when(c >= 3)
                def _free_ob():
                    drain_out(b)

                transpose_scale(b)
                start_out(c, b)
        return carry

    lax.fori_loop(0, (_UPW + 2) // 3, step, 0)
    drain_out(1)
    drain_out(2)
    drain_out(0)


def kernel(tokens, table):
    tok2d = tokens.astype(jnp.int32).T.reshape(_L * _NBH, 128)
    mesh = plsc.VectorSubcoreMesh(core_axis_name="c", subcore_axis_name="s")
    out5 = pl.kernel(
        _sc_embed,
        out_type=jax.ShapeDtypeStruct((_L, 8, _NBH, 8, 128), jnp.float32),
        mesh=mesh,
        scratch_types=[
            pltpu.VMEM((_TPU_ROWS, 128), jnp.int32),
            pltpu.VMEM((256, _EMB), jnp.float32),
            pltpu.VMEM((256, _EMB), jnp.float32),
            pltpu.VMEM((256, _EMB), jnp.float32),
            pltpu.VMEM((2, _EMB, 128), jnp.float32),
            pltpu.VMEM((2, _EMB, 128), jnp.float32),
            pltpu.VMEM((2, _EMB, 128), jnp.float32),
            pltpu.SemaphoreType.DMA((3,)),
            pltpu.SemaphoreType.DMA((3,)),
        ],
        compiler_params=pltpu.CompilerParams(
            use_tc_tiling_on_sc=False, needs_layout_passes=False
        ),
    )(tok2d, table)
    return out5.transpose((2, 4, 0, 1, 3)).reshape(_B, _L, _EMB)


# speedup vs baseline: 1.1209x; 1.0091x over previous
import jax
import jax.numpy as jnp
from jax import lax
from jax.experimental import pallas as pl
from jax.experimental.pallas import tpu as pltpu
from jax.experimental.pallas import tpu_sc as plsc

_EMB = 64
_SCALE = 8.0

_B = 4096
_L = 200
_N = _B * _L
_NW = 32
_NBH = _B // 128
_UPW = (_L * _NBH) // _NW


def _sc_embed(tok_hbm, table_hbm, out_hbm,
              idx_all, rows0, rows1, rows2, ob0, ob1, ob2, gsem, osem):
    wid = lax.axis_index("s") * 2 + lax.axis_index("c")
    u0 = wid * _UPW

    pltpu.sync_copy(tok_hbm.at[pl.ds(u0, _UPW)], idx_all)

    rows = (rows0, rows1, rows2)
    obs = (ob0, ob1, ob2)
    iota16 = lax.iota(jnp.int32, 16)
    rot = [lax.rem(iota16 + d, 16) for d in range(16)]

    def start_gather(c, b):
        pltpu.async_copy(
            table_hbm.at[idx_all.at[c]], rows[b], gsem.at[b]
        )

    def drain_gather(b):
        pltpu.make_async_copy(
            table_hbm.at[pl.ds(0, 128)], rows[b], gsem.at[b]
        ).wait()

    def drain_out(b):
        pltpu.make_async_copy(
            obs[b], table_hbm.at[pl.ds(0, 64)], osem.at[b]
        ).wait()

    def transpose_scale(b):
        rv = rows[b]
        ob = obs[b]

        def blkbody(u2, carry, ob=ob):
            tb = lax.shift_right_logical(u2, 2)
            eb = lax.bitwise_and(u2, 3)
            t0 = tb * 16
            e0 = eb * 16
            row_ids = iota16 + t0
            e_ids = [rot[d] + e0 for d in range(16)]
            vals = [plsc.load_gather(rv, [row_ids, e]) for e in e_ids]
            for d in range(16):
                plsc.store_scatter(ob, [e_ids[d], row_ids], vals[d] * _SCALE)
            return carry

        lax.fori_loop(0, 32, blkbody, 0)

    def start_out(c, b):
        u = u0 + c
        l = u // _NBH
        g = lax.rem(u, _NBH)
        for ehi in range(8):
            pltpu.async_copy(
                obs[b].at[pl.ds(8 * ehi, 8)],
                out_hbm.at[l, ehi, g],
                osem.at[b],
            )

    start_gather(0, 0)
    start_gather(1, 1)

    def step(k, carry):
        for b in range(3):
            c = k * 3 + b

            @pl.when(c < _UPW)
            def _unit():
                drain_gather(b)

                @pl.when(c + 2 < _UPW)
                def _prefetch():
                    start_gather(c + 2, (b + 2) % 3)

                @pl.when(c >= 3)
                def _free_ob():
                    drain_out(b)

                transpose_scale(b)
                start_out(c, b)
        return carry

    lax.fori_loop(0, (_UPW + 2) // 3, step, 0)
    drain_out(1)
    drain_out(2)
    drain_out(0)


def kernel(tokens, table):
    tok2d = tokens.astype(jnp.int32).T.reshape(_L * _NBH, 128)
    table128 = jnp.pad(table, ((0, 0), (0, 128 - _EMB)))
    mesh = plsc.VectorSubcoreMesh(core_axis_name="c", subcore_axis_name="s")
    out5 = pl.kernel(
        _sc_embed,
        out_type=jax.ShapeDtypeStruct((_L, 8, _NBH, 8, 128), jnp.float32),
        mesh=mesh,
        scratch_types=[
            pltpu.VMEM((_UPW, 128), jnp.int32),
            pltpu.VMEM((128, 128), jnp.float32),
            pltpu.VMEM((128, 128), jnp.float32),
            pltpu.VMEM((128, 128), jnp.float32),
            pltpu.VMEM((_EMB, 128), jnp.float32),
            pltpu.VMEM((_EMB, 128), jnp.float32),
            pltpu.VMEM((_EMB, 128), jnp.float32),
            pltpu.SemaphoreType.DMA((3,)),
            pltpu.SemaphoreType.DMA((3,)),
        ],
        compiler_params=pltpu.CompilerParams(
            use_tc_tiling_on_sc=False, needs_layout_passes=False
        ),
    )(tok2d, table128)
    return out5.transpose((2, 4, 0, 1, 3)).reshape(_B, _L, _EMB)


# speedup vs baseline: 1.2109x; 1.0803x over previous
import jax
import jax.numpy as jnp
from jax import lax
from jax.experimental import pallas as pl
from jax.experimental.pallas import tpu as pltpu
from jax.experimental.pallas import tpu_sc as plsc

_EMB = 64
_SCALE = 8.0

_B = 4096
_L = 200
_N = _B * _L
_NW = 32
_NBH = _B // 128
_UPW = (_L * _NBH) // _NW


def _sc_embed(tok_hbm, table_hbm, out_hbm,
              idx_all, idx2, rows0, rows1, rows2, ob0, ob1, ob2, gsem, osem):
    wid = lax.axis_index("s") * 2 + lax.axis_index("c")
    u0 = wid * _UPW

    pltpu.sync_copy(tok_hbm.at[pl.ds(u0, _UPW)], idx_all)

    rows = (rows0, rows1, rows2)
    obs = (ob0, ob1, ob2)
    iota16 = lax.iota(jnp.int32, 16)
    rot = [lax.rem(iota16 + d, 16) for d in range(16)]

    def start_gather(c, b):
        for k in range(8):
            sl = pl.ds(16 * k, 16)
            idx2[b, sl] = idx_all[c, sl] * 2
        pltpu.async_copy(
            table_hbm.at[idx2.at[b]], rows[b], gsem.at[b]
        )

    def drain_gather(b):
        pltpu.make_async_copy(
            table_hbm.at[pl.ds(0, 128)], rows[b], gsem.at[b]
        ).wait()

    def drain_out(b):
        pltpu.make_async_copy(
            rows[b], table_hbm.at[pl.ds(0, 128)], osem.at[b]
        ).wait()

    def transpose_scale(b):
        rv = rows[b]
        ob = obs[b]

        def blkbody(u2, carry, ob=ob):
            tb = lax.shift_right_logical(u2, 2)
            eb = lax.bitwise_and(u2, 3)
            t0 = tb * 16
            e0 = eb * 16
            row_ids = iota16 + t0
            e_ids = [rot[d] + e0 for d in range(16)]
            vals = [plsc.load_gather(rv, [row_ids, e]) for e in e_ids]
            for d in range(16):
                plsc.store_scatter(ob, [e_ids[d], row_ids], vals[d] * _SCALE)
            return carry

        lax.fori_loop(0, 32, blkbody, 0)

    def start_out(c, b):
        u = u0 + c
        l = u // _NBH
        g = lax.rem(u, _NBH)
        for ehi in range(8):
            pltpu.async_copy(
                obs[b].at[pl.ds(8 * ehi, 8)],
                out_hbm.at[l, ehi, g],
                osem.at[b],
            )

    start_gather(0, 0)
    start_gather(1, 1)

    def step(k, carry):
        for b in range(3):
            c = k * 3 + b

            @pl.when(c < _UPW)
            def _unit():
                drain_gather(b)

                @pl.when(c + 2 < _UPW)
                def _prefetch():
                    start_gather(c + 2, (b + 2) % 3)

                @pl.when(c >= 3)
                def _free_ob():
                    drain_out(b)

                transpose_scale(b)
                start_out(c, b)
        return carry

    lax.fori_loop(0, (_UPW + 2) // 3, step, 0)
    drain_out(1)
    drain_out(2)
    drain_out(0)


def kernel(tokens, table):
    tok2d = tokens.astype(jnp.int32).T.reshape(_L * _NBH, 128)
    table128 = jnp.pad(table, ((0, 0), (0, 128 - _EMB))).reshape(-1, _EMB)
    mesh = plsc.VectorSubcoreMesh(core_axis_name="c", subcore_axis_name="s")
    out5 = pl.kernel(
        _sc_embed,
        out_type=jax.ShapeDtypeStruct((_L, 8, _NBH, 8, 128), jnp.float32),
        mesh=mesh,
        scratch_types=[
            pltpu.VMEM((_UPW, 128), jnp.int32),
            pltpu.VMEM((3, 128), jnp.int32),
            pltpu.VMEM((128, _EMB), jnp.float32),
            pltpu.VMEM((128, _EMB), jnp.float32),
            pltpu.VMEM((128, _EMB), jnp.float32),
            pltpu.VMEM((_EMB, 128), jnp.float32),
            pltpu.VMEM((_EMB, 128), jnp.float32),
            pltpu.VMEM((_EMB, 128), jnp.float32),
            pltpu.SemaphoreType.DMA((3,)),
            pltpu.SemaphoreType.DMA((3,)),
        ],
        compiler_params=pltpu.CompilerParams(
            use_tc_tiling_on_sc=False, needs_layout_passes=False
        ),
    )(tok2d, table128)
    return out5.transpose((2, 4, 0, 1, 3)).reshape(_B, _L, _EMB)
